# Initial kernel scaffold; baseline (speedup 1.0000x reference)
import jax
import jax.numpy as jnp
from jax.experimental import pallas as pl


def kernel(x, edge_index, dist, W, b, k):
    raise NotImplementedError("write your pallas kernel here")


# speedup vs baseline: 5.8281x; 5.8281x over previous
import functools

import jax
import jax.numpy as jnp
from jax import lax
from jax.experimental import pallas as pl
from jax.experimental.pallas import tpu as pltpu
from jax.experimental.pallas import tpu_sc as plsc

N = 10000
E = 320000
D = 128
MIN_NORM = 1e-15
EPS = 4e-3

NC = 2
NS = 16
NW = NC * NS
EC = E // NW
CH = 80
NCHUNK = EC // CH
NPAD = 10240
RPT = NPAD // NS
ZR = 128
BR3 = 1024


def _artanh(x):
    x = jnp.clip(x, -1.0 + 1e-7, 1.0 - 1e-7)
    return 0.5 * (jnp.log1p(x) - jnp.log1p(-x))


def _norm(x):
    return jnp.maximum(
        jnp.sqrt(jnp.sum(x * x, axis=-1, keepdims=True)), MIN_NORM)


def _proj(x, c):
    norm = _norm(x)
    maxnorm = (1.0 - EPS) / jnp.sqrt(c)
    return jnp.where(norm > maxnorm, x / norm * maxnorm, x)


def _expmap0(u, c):
    sc = jnp.sqrt(c)
    un = _norm(u)
    return jnp.tanh(sc * un) * u / (sc * un)


def _logmap0(p, c):
    sc = jnp.sqrt(c)
    pn = _norm(p)
    return _artanh(sc * pn) * p / (sc * pn)


def _mobius_add(x, y, c):
    x2 = jnp.sum(x * x, axis=-1, keepdims=True)
    y2 = jnp.sum(y * y, axis=-1, keepdims=True)
    xy = jnp.sum(x * y, axis=-1, keepdims=True)
    num = (1.0 + 2.0 * c * xy + c * y2) * x + (1.0 - c * x2) * y
    denom = 1.0 + 2.0 * c * xy + c * c * x2 * y2
    return num / jnp.maximum(denom, MIN_NORM)


def _stage1_body(x_ref, w_ref, hb_ref, xt_ref):
    c = 1.0
    x = x_ref[...]
    w = w_ref[...]
    hb = hb_ref[...]
    xn = _norm(x)
    mx = lax.dot_general(x, w, (((1,), (1,)), ((), ())),
                         preferred_element_type=jnp.float32)
    mxn = _norm(mx)
    res = jnp.tanh(mxn / xn * _artanh(xn)) * mx / mxn
    cond = jnp.all(mx == 0.0, axis=-1, keepdims=True)
    res = jnp.where(cond, jnp.zeros_like(res), res)
    res = _proj(res, c)
    h = _proj(_mobius_add(res, hb, c), c)
    xt_ref[...] = _logmap0(h, c)


def _agg_body(xt_hbm, src_hbm, dst_hbm, agg_hbm, deg_hbm,
              src_idx, dst_idx, rows, onesv, zrow, zdeg, sem,
              agg_sh, deg_sh):
    cid = lax.axis_index("c")
    sid = lax.axis_index("s")
    wid = cid * NS + sid

    z16 = jnp.zeros((16,), jnp.float32)
    o16 = jnp.ones((16,), jnp.float32)

    def fill_zrow(i, _):
        r, q = i // (D // 16), i % (D // 16)
        zrow[r, pl.ds(q * 16, 16)] = z16
        return 0

    lax.fori_loop(0, ZR * (D // 16), fill_zrow, 0)

    def fill_zdeg(i, _):
        zdeg[pl.ds(i * 16, 16)] = z16
        return 0

    lax.fori_loop(0, RPT // 16, fill_zdeg, 0)

    for q in range(CH // 16):
        onesv[pl.ds(q * 16, 16)] = o16

    base_r = sid * RPT
    for m in range(RPT // ZR):
        pltpu.sync_copy(zrow, agg_sh.at[pl.ds(base_r + m * ZR, ZR)])
    pltpu.sync_copy(zdeg, deg_sh.at[pl.ds(base_r, RPT)])
    plsc.subcore_barrier()

    ebase = wid * EC

    def body(j, _):
        pltpu.sync_copy(src_hbm.at[pl.ds(ebase + j * CH, CH)], src_idx)
        pltpu.sync_copy(dst_hbm.at[pl.ds(ebase + j * CH, CH)], dst_idx)
        pltpu.async_copy(xt_hbm.at[src_idx], rows, sem).wait()
        pltpu.sync_copy(rows, agg_sh.at[dst_idx], add=True)
        pltpu.sync_copy(onesv, deg_sh.at[dst_idx], add=True)
        return 0

    lax.fori_loop(0, NCHUNK, body, 0)
    plsc.subcore_barrier()

    pltpu.sync_copy(agg_sh.at[pl.ds(base_r, RPT)],
                    agg_hbm.at[cid, pl.ds(base_r, RPT)])
    pltpu.sync_copy(deg_sh.at[pl.ds(base_r, RPT)],
                    deg_hbm.at[pl.ds(cid * NPAD + base_r, RPT)])


@functools.cache
def _agg_call():
    return pl.kernel(
        _agg_body,
        out_type=[
            jax.ShapeDtypeStruct((NC, NPAD, D), jnp.float32),
            jax.ShapeDtypeStruct((NC * NPAD,), jnp.float32),
        ],
        mesh=plsc.VectorSubcoreMesh(core_axis_name="c", subcore_axis_name="s",
                                    num_cores=NC, num_subcores=NS),
        scratch_types=[
            pltpu.VMEM((CH,), jnp.int32),
            pltpu.VMEM((CH,), jnp.int32),
            pltpu.VMEM((CH, D), jnp.float32),
            pltpu.VMEM((CH,), jnp.float32),
            pltpu.VMEM((ZR, D), jnp.float32),
            pltpu.VMEM((RPT,), jnp.float32),
            pltpu.SemaphoreType.DMA,
            pltpu.VMEM_SHARED((NPAD, D), jnp.float32),
            pltpu.VMEM_SHARED((NPAD,), jnp.float32),
        ],
    )


def _stage3_body(agg_ref, deg_ref, x_ref, k_ref, out_ref):
    agg2 = agg_ref[...]
    agg = agg2[0] + agg2[1]
    degf = deg_ref[...]
    deg8 = degf[0] + degf[1]
    r8 = lax.broadcasted_iota(jnp.int32, (BR3, 8), 0) // 128
    c8 = lax.broadcasted_iota(jnp.int32, (BR3, 8), 1)
    sel = (r8 == c8).astype(jnp.float32)
    brows = lax.dot_general(sel, deg8, (((1,), (0,)), ((), ())),
                            preferred_element_type=jnp.float32)
    l2 = lax.broadcasted_iota(jnp.int32, (BR3, D), 1)
    r2 = lax.broadcasted_iota(jnp.int32, (BR3, D), 0) % 128
    deg = jnp.sum(jnp.where(l2 == r2, brows, 0.0), axis=-1, keepdims=True)
    x = x_ref[...]
    k = k_ref[0]
    agg = agg / jnp.maximum(deg, 1.0)
    h = _proj(_expmap0(agg, 1.0), 1.0)
    xt = jax.nn.relu(_logmap0(h, 1.0))
    out1 = _proj(_expmap0(xt, 1.0), 1.0)
    out_ref[...] = _mobius_add(out1, k * x, 1.0)


def kernel(x, edge_index, dist, W, b, k):
    BR = 1000
    nblk = N // BR

    hyp_bias = _proj(_expmap0(b.reshape(1, -1), 1.0), 1.0)

    x_t = pl.pallas_call(
        _stage1_body,
        grid=(nblk,),
        in_specs=[
            pl.BlockSpec((BR, D), lambda i: (i, 0)),
            pl.BlockSpec((D, D), lambda i: (0, 0)),
            pl.BlockSpec((1, D), lambda i: (0, 0)),
        ],
        out_specs=pl.BlockSpec((BR, D), lambda i: (i, 0)),
        out_shape=jax.ShapeDtypeStruct((N, D), jnp.float32),
    )(x, W, hyp_bias)

    src = edge_index[0]
    dst = edge_index[1]
    agg_parts, deg_flat = _agg_call()(x_t, src, dst)
    deg_parts = deg_flat.reshape(NC, NPAD // 128, 128)

    out = pl.pallas_call(
        _stage3_body,
        grid=(NPAD // BR3,),
        in_specs=[
            pl.BlockSpec((NC, BR3, D), lambda i: (0, i, 0)),
            pl.BlockSpec((NC, BR3 // 128, 128), lambda i: (0, i, 0)),
            pl.BlockSpec((BR3, D), lambda i: (i, 0)),
            pl.BlockSpec(memory_space=pltpu.SMEM),
        ],
        out_specs=pl.BlockSpec((BR3, D), lambda i: (i, 0)),
        out_shape=jax.ShapeDtypeStruct((N, D), jnp.float32),
    )(agg_parts, deg_parts, x, k)

    return (out, edge_index, dist)


# speedup vs baseline: 9.5940x; 1.6462x over previous
import functools

import jax
import jax.numpy as jnp
from jax import lax
from jax.experimental import pallas as pl
from jax.experimental.pallas import tpu as pltpu
from jax.experimental.pallas import tpu_sc as plsc

N = 10000
E = 320000
D = 128
MIN_NORM = 1e-15
EPS = 4e-3

NC = 2
NS = 16
NW = NC * NS
EC = E // NW
CH = 40
NCHUNK = EC // CH
NPAD = 10240
RPT = NPAD // NS
ZR = 32
BR3 = 1024


def _artanh(x):
    x = jnp.clip(x, -1.0 + 1e-7, 1.0 - 1e-7)
    return 0.5 * (jnp.log1p(x) - jnp.log1p(-x))


def _norm(x):
    return jnp.maximum(
        jnp.sqrt(jnp.sum(x * x, axis=-1, keepdims=True)), MIN_NORM)


def _proj(x, c):
    norm = _norm(x)
    maxnorm = (1.0 - EPS) / jnp.sqrt(c)
    return jnp.where(norm > maxnorm, x / norm * maxnorm, x)


def _expmap0(u, c):
    sc = jnp.sqrt(c)
    un = _norm(u)
    return jnp.tanh(sc * un) * u / (sc * un)


def _logmap0(p, c):
    sc = jnp.sqrt(c)
    pn = _norm(p)
    return _artanh(sc * pn) * p / (sc * pn)


def _mobius_add(x, y, c):
    x2 = jnp.sum(x * x, axis=-1, keepdims=True)
    y2 = jnp.sum(y * y, axis=-1, keepdims=True)
    xy = jnp.sum(x * y, axis=-1, keepdims=True)
    num = (1.0 + 2.0 * c * xy + c * y2) * x + (1.0 - c * x2) * y
    denom = 1.0 + 2.0 * c * xy + c * c * x2 * y2
    return num / jnp.maximum(denom, MIN_NORM)


def _stage1_body(x_ref, w_ref, hb_ref, xt_ref):
    c = 1.0
    x = x_ref[...]
    w = w_ref[...]
    hb = hb_ref[...]
    xn = _norm(x)
    mx = lax.dot_general(x, w, (((1,), (1,)), ((), ())),
                         preferred_element_type=jnp.float32)
    mxn = _norm(mx)
    res = jnp.tanh(mxn / xn * _artanh(xn)) * mx / mxn
    cond = jnp.all(mx == 0.0, axis=-1, keepdims=True)
    res = jnp.where(cond, jnp.zeros_like(res), res)
    res = _proj(res, c)
    h = _proj(_mobius_add(res, hb, c), c)
    xt_ref[...] = _logmap0(h, c)


NBUF = 5
NGROUP = NCHUNK // NBUF


def _agg_body(xt_hbm, src_hbm, dst_hbm, agg_hbm, deg_hbm,
              src_idx, dst_idx, rows, onesv, zrow,
              sem_i, sem_g, sem_s, sem_d,
              agg_sh, deg_sh):
    cid = lax.axis_index("c")
    sid = lax.axis_index("s")
    wid = cid * NS + sid

    z16 = jnp.zeros((16,), jnp.float32)
    o16 = jnp.ones((16,), jnp.float32)

    def fill_zrow(i, _):
        r, q = i // (D // 16), i % (D // 16)
        zrow[r, pl.ds(q * 16, 16)] = z16
        return 0

    lax.fori_loop(0, ZR * (D // 16), fill_zrow, 0)

    for o in (0, 16, CH - 16):
        onesv[pl.ds(o, 16)] = o16

    base_r = sid * RPT
    for m in range(RPT // ZR):
        pltpu.sync_copy(zrow, agg_sh.at[pl.ds(base_r + m * ZR, ZR)])
    for m in range(RPT // D):
        pltpu.sync_copy(zrow.at[0], deg_sh.at[pl.ds(base_r + m * D, D)])
    plsc.subcore_barrier()

    ebase = wid * EC

    def idx_pair(j, p, b):
        pltpu.async_copy(src_hbm.at[pl.ds(ebase + j * CH, CH)],
                         src_idx.at[p, b], sem_i)
        pltpu.async_copy(dst_hbm.at[pl.ds(ebase + j * CH, CH)],
                         dst_idx.at[p, b], sem_i)

    def drain_idx(n):
        for _ in range(n):
            pltpu.make_async_copy(src_hbm.at[pl.ds(0, CH)],
                                  src_idx.at[0, 0], sem_i).wait()

    def drain_rows(n, sem):
        for _ in range(n):
            pltpu.make_async_copy(xt_hbm.at[pl.ds(0, CH)],
                                  rows.at[0], sem).wait()

    def drain_deg(n):
        for _ in range(n):
            pltpu.make_async_copy(deg_hbm.at[pl.ds(0, CH)], onesv,
                                  sem_d).wait()

    for b in range(NBUF):
        idx_pair(b, 0, b)

    def group(t, _):
        p = t % 2

        drain_idx(2 * NBUF)
        for b in range(NBUF):
            pltpu.async_copy(xt_hbm.at[src_idx.at[p, b]], rows.at[b], sem_g)

        @pl.when(t < NGROUP - 1)
        def _():
            for b in range(NBUF):
                idx_pair((t + 1) * NBUF + b, 1 - p, b)

        drain_rows(NBUF, sem_g)
        for b in range(NBUF):
            pltpu.async_copy(rows.at[b], agg_sh.at[dst_idx.at[p, b]],
                             sem_s, add=True)
            pltpu.async_copy(onesv, deg_sh.at[dst_idx.at[p, b]],
                             sem_d, add=True)
        drain_rows(NBUF, sem_s)
        drain_deg(NBUF)
        return 0

    lax.fori_loop(0, NGROUP, group, 0)
    plsc.subcore_barrier()

    pltpu.sync_copy(agg_sh.at[pl.ds(base_r, RPT)],
                    agg_hbm.at[cid, pl.ds(base_r, RPT)])
    pltpu.sync_copy(deg_sh.at[pl.ds(base_r, RPT)],
                    deg_hbm.at[pl.ds(cid * NPAD + base_r, RPT)])


@functools.cache
def _agg_call():
    return pl.kernel(
        _agg_body,
        out_type=[
            jax.ShapeDtypeStruct((NC, NPAD, D), jnp.float32),
            jax.ShapeDtypeStruct((NC * NPAD,), jnp.float32),
        ],
        mesh=plsc.VectorSubcoreMesh(core_axis_name="c", subcore_axis_name="s",
                                    num_cores=NC, num_subcores=NS),
        scratch_types=[
            pltpu.VMEM((2, NBUF, CH), jnp.int32),
            pltpu.VMEM((2, NBUF, CH), jnp.int32),
            pltpu.VMEM((NBUF, CH, D), jnp.float32),
            pltpu.VMEM((CH,), jnp.float32),
            pltpu.VMEM((ZR, D), jnp.float32),
            pltpu.SemaphoreType.DMA,
            pltpu.SemaphoreType.DMA,
            pltpu.SemaphoreType.DMA,
            pltpu.SemaphoreType.DMA,
            pltpu.VMEM_SHARED((NPAD, D), jnp.float32),
            pltpu.VMEM_SHARED((NPAD,), jnp.float32),
        ],
    )


def _stage3_body(agg_ref, deg_ref, x_ref, k_ref, out_ref):
    agg2 = agg_ref[...]
    agg = agg2[0] + agg2[1]
    degf = deg_ref[...]
    deg8 = degf[0] + degf[1]
    r8 = lax.broadcasted_iota(jnp.int32, (BR3, 8), 0) // 128
    c8 = lax.broadcasted_iota(jnp.int32, (BR3, 8), 1)
    sel = (r8 == c8).astype(jnp.float32)
    brows = lax.dot_general(sel, deg8, (((1,), (0,)), ((), ())),
                            preferred_element_type=jnp.float32)
    l2 = lax.broadcasted_iota(jnp.int32, (BR3, D), 1)
    r2 = lax.broadcasted_iota(jnp.int32, (BR3, D), 0) % 128
    deg = jnp.sum(jnp.where(l2 == r2, brows, 0.0), axis=-1, keepdims=True)
    x = x_ref[...]
    k = k_ref[0]
    agg = agg / jnp.maximum(deg, 1.0)
    h = _proj(_expmap0(agg, 1.0), 1.0)
    xt = jax.nn.relu(_logmap0(h, 1.0))
    out1 = _proj(_expmap0(xt, 1.0), 1.0)
    out_ref[...] = _mobius_add(out1, k * x, 1.0)


def kernel(x, edge_index, dist, W, b, k):
    BR = 1000
    nblk = N // BR

    hyp_bias = _proj(_expmap0(b.reshape(1, -1), 1.0), 1.0)

    x_t = pl.pallas_call(
        _stage1_body,
        grid=(nblk,),
        in_specs=[
            pl.BlockSpec((BR, D), lambda i: (i, 0)),
            pl.BlockSpec((D, D), lambda i: (0, 0)),
            pl.BlockSpec((1, D), lambda i: (0, 0)),
        ],
        out_specs=pl.BlockSpec((BR, D), lambda i: (i, 0)),
        out_shape=jax.ShapeDtypeStruct((N, D), jnp.float32),
    )(x, W, hyp_bias)

    src = edge_index[0]
    dst = edge_index[1]
    agg_parts, deg_flat = _agg_call()(x_t, src, dst)
    deg_parts = deg_flat.reshape(NC, NPAD // 128, 128)

    out = pl.pallas_call(
        _stage3_body,
        grid=(NPAD // BR3,),
        in_specs=[
            pl.BlockSpec((NC, BR3, D), lambda i: (0, i, 0)),
            pl.BlockSpec((NC, BR3 // 128, 128), lambda i: (0, i, 0)),
            pl.BlockSpec((BR3, D), lambda i: (i, 0)),
            pl.BlockSpec(memory_space=pltpu.SMEM),
        ],
        out_specs=pl.BlockSpec((BR3, D), lambda i: (i, 0)),
        out_shape=jax.ShapeDtypeStruct((N, D), jnp.float32),
    )(agg_parts, deg_parts, x, k)

    return (out, edge_index, dist)


# speedup vs baseline: 11.6674x; 1.2161x over previous
import functools

import jax
import jax.numpy as jnp
from jax import lax
from jax.experimental import pallas as pl
from jax.experimental.pallas import tpu as pltpu
from jax.experimental.pallas import tpu_sc as plsc

N = 10000
E = 320000
D = 128
MIN_NORM = 1e-15
EPS = 4e-3

NC = 2
NS = 16
NW = NC * NS
EC = E // NW
CH = 40
NCHUNK = EC // CH
NPAD = 10240
RPT = NPAD // NS
ZR = 32
BR3 = 1024


def _artanh(x):
    x = jnp.clip(x, -1.0 + 1e-7, 1.0 - 1e-7)
    return 0.5 * (jnp.log1p(x) - jnp.log1p(-x))


def _norm(x):
    return jnp.maximum(
        jnp.sqrt(jnp.sum(x * x, axis=-1, keepdims=True)), MIN_NORM)


def _proj(x, c):
    norm = _norm(x)
    maxnorm = (1.0 - EPS) / jnp.sqrt(c)
    return jnp.where(norm > maxnorm, x / norm * maxnorm, x)


def _expmap0(u, c):
    sc = jnp.sqrt(c)
    un = _norm(u)
    return jnp.tanh(sc * un) * u / (sc * un)


def _logmap0(p, c):
    sc = jnp.sqrt(c)
    pn = _norm(p)
    return _artanh(sc * pn) * p / (sc * pn)


def _mobius_add(x, y, c):
    x2 = jnp.sum(x * x, axis=-1, keepdims=True)
    y2 = jnp.sum(y * y, axis=-1, keepdims=True)
    xy = jnp.sum(x * y, axis=-1, keepdims=True)
    num = (1.0 + 2.0 * c * xy + c * y2) * x + (1.0 - c * x2) * y
    denom = 1.0 + 2.0 * c * xy + c * c * x2 * y2
    return num / jnp.maximum(denom, MIN_NORM)


def _stage1_body(x_ref, w_ref, hb_ref, xt_ref):
    c = 1.0
    x = x_ref[...]
    w = w_ref[...]
    hb = hb_ref[...]
    xn = _norm(x)
    mx = lax.dot_general(x, w, (((1,), (1,)), ((), ())),
                         preferred_element_type=jnp.float32)
    mxn = _norm(mx)
    res = jnp.tanh(mxn / xn * _artanh(xn)) * mx / mxn
    cond = jnp.all(mx == 0.0, axis=-1, keepdims=True)
    res = jnp.where(cond, jnp.zeros_like(res), res)
    res = _proj(res, c)
    h = _proj(_mobius_add(res, hb, c), c)
    xt_ref[...] = _logmap0(h, c)


NBUF = 5
NGROUP = NCHUNK // NBUF


def _agg_body(xt_hbm, src_hbm, dst_hbm, agg_hbm, deg_hbm,
              src_idx, dst_idx, rows, onesv, zrow,
              sem_i, g0, g1, g2, g3, g4, s0, s1, s2, s3, s4,
              agg_sh, deg_sh):
    gsem = [g0, g1, g2, g3, g4]
    ssem = [s0, s1, s2, s3, s4]
    cid = lax.axis_index("c")
    sid = lax.axis_index("s")
    wid = cid * NS + sid

    z16 = jnp.zeros((16,), jnp.float32)
    o16 = jnp.ones((16,), jnp.float32)

    def fill_zrow(i, _):
        r, q = i // (D // 16), i % (D // 16)
        zrow[r, pl.ds(q * 16, 16)] = z16
        return 0

    lax.fori_loop(0, ZR * (D // 16), fill_zrow, 0)

    for o in (0, 16, CH - 16):
        onesv[pl.ds(o, 16)] = o16

    base_r = sid * RPT
    for m in range(RPT // ZR):
        pltpu.async_copy(zrow, agg_sh.at[pl.ds(base_r + m * ZR, ZR)], sem_i)
    for m in range(RPT // D):
        pltpu.async_copy(zrow.at[0], deg_sh.at[pl.ds(base_r + m * D, D)],
                         sem_i)
    for m in range(RPT // ZR):
        pltpu.make_async_copy(zrow, agg_sh.at[pl.ds(base_r, ZR)],
                              sem_i).wait()
    for m in range(RPT // D):
        pltpu.make_async_copy(zrow.at[0], deg_sh.at[pl.ds(base_r, D)],
                              sem_i).wait()
    plsc.subcore_barrier()

    ebase = wid * EC

    def idx_pair(j, p, b):
        pltpu.async_copy(src_hbm.at[pl.ds(ebase + j * CH, CH)],
                         src_idx.at[p, b], sem_i)
        pltpu.async_copy(dst_hbm.at[pl.ds(ebase + j * CH, CH)],
                         dst_idx.at[p, b], sem_i)

    def drain_idx(n):
        for _ in range(n):
            pltpu.make_async_copy(src_hbm.at[pl.ds(0, CH)],
                                  src_idx.at[0, 0], sem_i).wait()

    def wait_gather(b):
        pltpu.make_async_copy(xt_hbm.at[pl.ds(0, CH)], rows.at[b],
                              gsem[b]).wait()

    def wait_scatter(b):
        pltpu.make_async_copy(xt_hbm.at[pl.ds(0, CH)], rows.at[b],
                              ssem[b]).wait()
        pltpu.make_async_copy(deg_hbm.at[pl.ds(0, CH)], onesv,
                              ssem[b]).wait()

    for b in range(NBUF):
        idx_pair(b, 0, b)

    def group(t, _):
        p = t % 2

        drain_idx(2 * NBUF)
        for b in range(NBUF):
            @pl.when(t > 0)
            def _(b=b):
                wait_scatter(b)

            pltpu.async_copy(xt_hbm.at[src_idx.at[p, b]], rows.at[b],
                             gsem[b])

        @pl.when(t < NGROUP - 1)
        def _():
            for b in range(NBUF):
                idx_pair((t + 1) * NBUF + b, 1 - p, b)

        for b in range(NBUF):
            wait_gather(b)
            pltpu.async_copy(rows.at[b], agg_sh.at[dst_idx.at[p, b]],
                             ssem[b], add=True)
            pltpu.async_copy(onesv, deg_sh.at[dst_idx.at[p, b]],
                             ssem[b], add=True)
        return 0

    lax.fori_loop(0, NGROUP, group, 0)
    for b in range(NBUF):
        wait_scatter(b)
    plsc.subcore_barrier()

    pltpu.sync_copy(agg_sh.at[pl.ds(base_r, RPT)],
                    agg_hbm.at[cid, pl.ds(base_r, RPT)])
    pltpu.sync_copy(deg_sh.at[pl.ds(base_r, RPT)],
                    deg_hbm.at[pl.ds(cid * NPAD + base_r, RPT)])


@functools.cache
def _agg_call():
    return pl.kernel(
        _agg_body,
        out_type=[
            jax.ShapeDtypeStruct((NC, NPAD, D), jnp.float32),
            jax.ShapeDtypeStruct((NC * NPAD,), jnp.float32),
        ],
        mesh=plsc.VectorSubcoreMesh(core_axis_name="c", subcore_axis_name="s",
                                    num_cores=NC, num_subcores=NS),
        scratch_types=[
            pltpu.VMEM((2, NBUF, CH), jnp.int32),
            pltpu.VMEM((2, NBUF, CH), jnp.int32),
            pltpu.VMEM((NBUF, CH, D), jnp.float32),
            pltpu.VMEM((CH,), jnp.float32),
            pltpu.VMEM((ZR, D), jnp.float32),
            pltpu.SemaphoreType.DMA,
            pltpu.SemaphoreType.DMA,
            pltpu.SemaphoreType.DMA,
            pltpu.SemaphoreType.DMA,
            pltpu.SemaphoreType.DMA,
            pltpu.SemaphoreType.DMA,
            pltpu.SemaphoreType.DMA,
            pltpu.SemaphoreType.DMA,
            pltpu.SemaphoreType.DMA,
            pltpu.SemaphoreType.DMA,
            pltpu.SemaphoreType.DMA,
            pltpu.VMEM_SHARED((NPAD, D), jnp.float32),
            pltpu.VMEM_SHARED((NPAD,), jnp.float32),
        ],
    )


def _stage3_body(agg_ref, deg_ref, x_ref, k_ref, out_ref):
    agg2 = agg_ref[...]
    agg = agg2[0] + agg2[1]
    degf = deg_ref[...]
    deg8 = degf[0] + degf[1]
    r8 = lax.broadcasted_iota(jnp.int32, (BR3, 8), 0) // 128
    c8 = lax.broadcasted_iota(jnp.int32, (BR3, 8), 1)
    sel = (r8 == c8).astype(jnp.float32)
    brows = lax.dot_general(sel, deg8, (((1,), (0,)), ((), ())),
                            preferred_element_type=jnp.float32)
    l2 = lax.broadcasted_iota(jnp.int32, (BR3, D), 1)
    r2 = lax.broadcasted_iota(jnp.int32, (BR3, D), 0) % 128
    deg = jnp.sum(jnp.where(l2 == r2, brows, 0.0), axis=-1, keepdims=True)
    x = x_ref[...]
    k = k_ref[0]
    agg = agg / jnp.maximum(deg, 1.0)
    h = _proj(_expmap0(agg, 1.0), 1.0)
    xt = jax.nn.relu(_logmap0(h, 1.0))
    out1 = _proj(_expmap0(xt, 1.0), 1.0)
    out_ref[...] = _mobius_add(out1, k * x, 1.0)


def kernel(x, edge_index, dist, W, b, k):
    BR = 1000
    nblk = N // BR

    hyp_bias = _proj(_expmap0(b.reshape(1, -1), 1.0), 1.0)

    x_t = pl.pallas_call(
        _stage1_body,
        grid=(nblk,),
        in_specs=[
            pl.BlockSpec((BR, D), lambda i: (i, 0)),
            pl.BlockSpec((D, D), lambda i: (0, 0)),
            pl.BlockSpec((1, D), lambda i: (0, 0)),
        ],
        out_specs=pl.BlockSpec((BR, D), lambda i: (i, 0)),
        out_shape=jax.ShapeDtypeStruct((N, D), jnp.float32),
    )(x, W, hyp_bias)

    src = edge_index[0]
    dst = edge_index[1]
    agg_parts, deg_flat = _agg_call()(x_t, src, dst)
    deg_parts = deg_flat.reshape(NC, NPAD // 128, 128)

    out = pl.pallas_call(
        _stage3_body,
        grid=(NPAD // BR3,),
        in_specs=[
            pl.BlockSpec((NC, BR3, D), lambda i: (0, i, 0)),
            pl.BlockSpec((NC, BR3 // 128, 128), lambda i: (0, i, 0)),
            pl.BlockSpec((BR3, D), lambda i: (i, 0)),
            pl.BlockSpec(memory_space=pltpu.SMEM),
        ],
        out_specs=pl.BlockSpec((BR3, D), lambda i: (i, 0)),
        out_shape=jax.ShapeDtypeStruct((N, D), jnp.float32),
    )(agg_parts, deg_parts, x, k)

    return (out, edge_index, dist)


# speedup vs baseline: 12.7636x; 1.0939x over previous
import functools

import jax
import jax.numpy as jnp
from jax import lax
from jax.experimental import pallas as pl
from jax.experimental.pallas import tpu as pltpu
from jax.experimental.pallas import tpu_sc as plsc

N = 10000
E = 320000
D = 128
MIN_NORM = 1e-15
EPS = 4e-3

NC = 2
NS = 16
NW = NC * NS
EC = E // NW
CH = 40
NCHUNK = EC // CH
NPAD = 10240
RPT = NPAD // NS
ZR = 32
BR3 = 1024


def _artanh(x):
    x = jnp.clip(x, -1.0 + 1e-7, 1.0 - 1e-7)
    return 0.5 * (jnp.log1p(x) - jnp.log1p(-x))


def _norm(x):
    return jnp.maximum(
        jnp.sqrt(jnp.sum(x * x, axis=-1, keepdims=True)), MIN_NORM)


def _proj(x, c):
    norm = _norm(x)
    maxnorm = (1.0 - EPS) / jnp.sqrt(c)
    return jnp.where(norm > maxnorm, x / norm * maxnorm, x)


def _expmap0(u, c):
    sc = jnp.sqrt(c)
    un = _norm(u)
    return jnp.tanh(sc * un) * u / (sc * un)


def _logmap0(p, c):
    sc = jnp.sqrt(c)
    pn = _norm(p)
    return _artanh(sc * pn) * p / (sc * pn)


def _mobius_add(x, y, c):
    x2 = jnp.sum(x * x, axis=-1, keepdims=True)
    y2 = jnp.sum(y * y, axis=-1, keepdims=True)
    xy = jnp.sum(x * y, axis=-1, keepdims=True)
    num = (1.0 + 2.0 * c * xy + c * y2) * x + (1.0 - c * x2) * y
    denom = 1.0 + 2.0 * c * xy + c * c * x2 * y2
    return num / jnp.maximum(denom, MIN_NORM)


def _stage1_body(x_ref, w_ref, b_ref, e_ref, xt_ref, src_ref, dst_ref):
    c = 1.0
    x = x_ref[...]
    w = w_ref[...]
    hb = _proj(_expmap0(b_ref[...], c), c)
    xn = _norm(x)
    mx = lax.dot_general(x, w, (((1,), (1,)), ((), ())),
                         preferred_element_type=jnp.float32)
    mxn = _norm(mx)
    res = jnp.tanh(mxn / xn * _artanh(xn)) * mx / mxn
    cond = jnp.all(mx == 0.0, axis=-1, keepdims=True)
    res = jnp.where(cond, jnp.zeros_like(res), res)
    res = _proj(res, c)
    h = _proj(_mobius_add(res, hb, c), c)
    xt_ref[...] = _logmap0(h, c)
    @pl.when(pl.program_id(0) == 0)
    def _():
        e2 = e_ref[...]
        src_ref[...] = e2[0]
        dst_ref[...] = e2[1]


NBUF = 5
NGROUP = NCHUNK // NBUF


def _agg_body(xt_hbm, src_hbm, dst_hbm, agg_hbm, deg_hbm,
              src_idx, dst_idx, rows, onesv, zrow,
              sem_i, g0, g1, g2, g3, g4, s0, s1, s2, s3, s4,
              agg_sh, deg_sh):
    gsem = [g0, g1, g2, g3, g4]
    ssem = [s0, s1, s2, s3, s4]
    cid = lax.axis_index("c")
    sid = lax.axis_index("s")
    wid = cid * NS + sid

    z16 = jnp.zeros((16,), jnp.float32)
    o16 = jnp.ones((16,), jnp.float32)

    def fill_zrow(i, _):
        r, q = i // (D // 16), i % (D // 16)
        zrow[r, pl.ds(q * 16, 16)] = z16
        return 0

    lax.fori_loop(0, ZR * (D // 16), fill_zrow, 0)

    for o in (0, 16, CH - 16):
        onesv[pl.ds(o, 16)] = o16

    base_r = sid * RPT
    for m in range(RPT // ZR):
        pltpu.async_copy(zrow, agg_sh.at[pl.ds(base_r + m * ZR, ZR)], g0)
    for m in range(RPT // D):
        pltpu.async_copy(zrow.at[0], deg_sh.at[pl.ds(base_r + m * D, D)], g0)

    ebase = wid * EC

    def idx_pair(j, p, b):
        pltpu.async_copy(src_hbm.at[pl.ds(ebase + j * CH, CH)],
                         src_idx.at[p, b], sem_i)
        pltpu.async_copy(dst_hbm.at[pl.ds(ebase + j * CH, CH)],
                         dst_idx.at[p, b], sem_i)

    def drain_idx(n):
        for _ in range(n):
            pltpu.make_async_copy(src_hbm.at[pl.ds(0, CH)],
                                  src_idx.at[0, 0], sem_i).wait()

    def wait_gather(b):
        pltpu.make_async_copy(xt_hbm.at[pl.ds(0, CH)], rows.at[b],
                              gsem[b]).wait()

    def wait_scatter(b):
        pltpu.make_async_copy(xt_hbm.at[pl.ds(0, CH)], rows.at[b],
                              ssem[b]).wait()
        pltpu.make_async_copy(deg_hbm.at[pl.ds(0, CH)], onesv,
                              ssem[b]).wait()

    for b in range(NBUF):
        idx_pair(b, 0, b)
    for m in range(RPT // ZR):
        pltpu.make_async_copy(zrow, agg_sh.at[pl.ds(base_r, ZR)], g0).wait()
    for m in range(RPT // D):
        pltpu.make_async_copy(zrow.at[0], deg_sh.at[pl.ds(base_r, D)],
                              g0).wait()
    plsc.subcore_barrier()

    def group(t, _):
        p = t % 2

        drain_idx(2 * NBUF)
        for b in range(NBUF):
            @pl.when(t > 0)
            def _(b=b):
                wait_scatter(b)

            pltpu.async_copy(xt_hbm.at[src_idx.at[p, b]], rows.at[b],
                             gsem[b])

        @pl.when(t < NGROUP - 1)
        def _():
            for b in range(NBUF):
                idx_pair((t + 1) * NBUF + b, 1 - p, b)

        for b in range(NBUF):
            wait_gather(b)
            pltpu.async_copy(rows.at[b], agg_sh.at[dst_idx.at[p, b]],
                             ssem[b], add=True)
            pltpu.async_copy(onesv, deg_sh.at[dst_idx.at[p, b]],
                             ssem[b], add=True)
        return 0

    lax.fori_loop(0, NGROUP, group, 0)
    for b in range(NBUF):
        wait_scatter(b)
    plsc.subcore_barrier()

    pltpu.sync_copy(agg_sh.at[pl.ds(base_r, RPT)],
                    agg_hbm.at[cid, pl.ds(base_r, RPT)])
    pltpu.sync_copy(deg_sh.at[pl.ds(base_r, RPT)],
                    deg_hbm.at[pl.ds(cid * NPAD + base_r, RPT)])


@functools.cache
def _agg_call():
    return pl.kernel(
        _agg_body,
        out_type=[
            jax.ShapeDtypeStruct((NC, NPAD, D), jnp.float32),
            jax.ShapeDtypeStruct((NC * NPAD,), jnp.float32),
        ],
        mesh=plsc.VectorSubcoreMesh(core_axis_name="c", subcore_axis_name="s",
                                    num_cores=NC, num_subcores=NS),
        scratch_types=[
            pltpu.VMEM((2, NBUF, CH), jnp.int32),
            pltpu.VMEM((2, NBUF, CH), jnp.int32),
            pltpu.VMEM((NBUF, CH, D), jnp.float32),
            pltpu.VMEM((CH,), jnp.float32),
            pltpu.VMEM((ZR, D), jnp.float32),
            pltpu.SemaphoreType.DMA,
            pltpu.SemaphoreType.DMA,
            pltpu.SemaphoreType.DMA,
            pltpu.SemaphoreType.DMA,
            pltpu.SemaphoreType.DMA,
            pltpu.SemaphoreType.DMA,
            pltpu.SemaphoreType.DMA,
            pltpu.SemaphoreType.DMA,
            pltpu.SemaphoreType.DMA,
            pltpu.SemaphoreType.DMA,
            pltpu.SemaphoreType.DMA,
            pltpu.VMEM_SHARED((NPAD, D), jnp.float32),
            pltpu.VMEM_SHARED((NPAD,), jnp.float32),
        ],
    )


def _stage3_body(agg_ref, deg_ref, x_ref, k_ref, out_ref):
    agg2 = agg_ref[...]
    agg = agg2[0] + agg2[1]
    degf = deg_ref[...]
    deg8 = degf[0] + degf[1]
    r8 = lax.broadcasted_iota(jnp.int32, (BR3, 8), 0) // 128
    c8 = lax.broadcasted_iota(jnp.int32, (BR3, 8), 1)
    sel = (r8 == c8).astype(jnp.float32)
    brows = lax.dot_general(sel, deg8, (((1,), (0,)), ((), ())),
                            preferred_element_type=jnp.float32)
    l2 = lax.broadcasted_iota(jnp.int32, (BR3, D), 1)
    r2 = lax.broadcasted_iota(jnp.int32, (BR3, D), 0) % 128
    deg = jnp.sum(jnp.where(l2 == r2, brows, 0.0), axis=-1, keepdims=True)
    x = x_ref[...]
    k = k_ref[0]
    agg = agg / jnp.maximum(deg, 1.0)
    h = _proj(_expmap0(agg, 1.0), 1.0)
    xt = jax.nn.relu(_logmap0(h, 1.0))
    out1 = _proj(_expmap0(xt, 1.0), 1.0)
    out_ref[...] = _mobius_add(out1, k * x, 1.0)


def kernel(x, edge_index, dist, W, b, k):
    BR = 1000
    nblk = N // BR

    x_t, src, dst = pl.pallas_call(
        _stage1_body,
        grid=(nblk,),
        in_specs=[
            pl.BlockSpec((BR, D), lambda i: (i, 0)),
            pl.BlockSpec((D, D), lambda i: (0, 0)),
            pl.BlockSpec((1, D), lambda i: (0, 0)),
            pl.BlockSpec((2, E), lambda i: (0, 0)),
        ],
        out_specs=[
            pl.BlockSpec((BR, D), lambda i: (i, 0)),
            pl.BlockSpec((E,), lambda i: (0,)),
            pl.BlockSpec((E,), lambda i: (0,)),
        ],
        out_shape=[
            jax.ShapeDtypeStruct((N, D), jnp.float32),
            jax.ShapeDtypeStruct((E,), jnp.int32),
            jax.ShapeDtypeStruct((E,), jnp.int32),
        ],
    )(x, W, b.reshape(1, -1), edge_index)

    agg_parts, deg_flat = _agg_call()(x_t, src, dst)
    deg_parts = deg_flat.reshape(NC, NPAD // 128, 128)

    out = pl.pallas_call(
        _stage3_body,
        grid=(NPAD // BR3,),
        in_specs=[
            pl.BlockSpec((NC, BR3, D), lambda i: (0, i, 0)),
            pl.BlockSpec((NC, BR3 // 128, 128), lambda i: (0, i, 0)),
            pl.BlockSpec((BR3, D), lambda i: (i, 0)),
            pl.BlockSpec(memory_space=pltpu.SMEM),
        ],
        out_specs=pl.BlockSpec((BR3, D), lambda i: (i, 0)),
        out_shape=jax.ShapeDtypeStruct((N, D), jnp.float32),
    )(agg_parts, deg_parts, x, k)

    return (out, edge_index, dist)


# speedup vs baseline: 12.7647x; 1.0001x over previous
import functools

import jax
import jax.numpy as jnp
from jax import lax
from jax.experimental import pallas as pl
from jax.experimental.pallas import tpu as pltpu
from jax.experimental.pallas import tpu_sc as plsc

N = 10000
E = 320000
D = 128
MIN_NORM = 1e-15
EPS = 4e-3

NC = 2
NS = 16
NW = NC * NS
EC = E // NW
CH = 40
NCHUNK = EC // CH
NPAD = 10240
RPT = NPAD // NS
ZR = 32
BR3 = 1024


def _artanh(x):
    x = jnp.clip(x, -1.0 + 1e-7, 1.0 - 1e-7)
    return 0.5 * (jnp.log1p(x) - jnp.log1p(-x))


def _norm(x):
    return jnp.maximum(
        jnp.sqrt(jnp.sum(x * x, axis=-1, keepdims=True)), MIN_NORM)


def _proj(x, c):
    norm = _norm(x)
    maxnorm = (1.0 - EPS) / jnp.sqrt(c)
    return jnp.where(norm > maxnorm, x / norm * maxnorm, x)


def _expmap0(u, c):
    sc = jnp.sqrt(c)
    un = _norm(u)
    return jnp.tanh(sc * un) * u / (sc * un)


def _logmap0(p, c):
    sc = jnp.sqrt(c)
    pn = _norm(p)
    return _artanh(sc * pn) * p / (sc * pn)


def _mobius_add(x, y, c):
    x2 = jnp.sum(x * x, axis=-1, keepdims=True)
    y2 = jnp.sum(y * y, axis=-1, keepdims=True)
    xy = jnp.sum(x * y, axis=-1, keepdims=True)
    num = (1.0 + 2.0 * c * xy + c * y2) * x + (1.0 - c * x2) * y
    denom = 1.0 + 2.0 * c * xy + c * c * x2 * y2
    return num / jnp.maximum(denom, MIN_NORM)


def _stage1_body(x_ref, w_ref, b_ref, e_ref, xt_ref, src_ref, dst_ref):
    c = 1.0
    x = x_ref[...]
    w = w_ref[...]
    hb = _proj(_expmap0(b_ref[...], c), c)
    xn = _norm(x)
    mx = lax.dot_general(x, w, (((1,), (1,)), ((), ())),
                         preferred_element_type=jnp.float32)
    mxn = _norm(mx)
    res = jnp.tanh(mxn / xn * _artanh(xn)) * mx / mxn
    cond = jnp.all(mx == 0.0, axis=-1, keepdims=True)
    res = jnp.where(cond, jnp.zeros_like(res), res)
    res = _proj(res, c)
    h = _proj(_mobius_add(res, hb, c), c)
    xt_ref[...] = _logmap0(h, c)
    @pl.when(pl.program_id(0) == 0)
    def _():
        e2 = e_ref[...]
        src_ref[...] = e2[0]
        dst_ref[...] = e2[1]


NBUF = 5
NGROUP = NCHUNK // NBUF


def _agg_body(xt_hbm, src_hbm, dst_hbm, agg_hbm, deg_hbm,
              src_idx, dst_idx, rows, onesv, zrow,
              sem_i, g0, g1, g2, g3, g4, s0, s1, s2, s3, s4,
              agg_sh, deg_sh):
    gsem = [g0, g1, g2, g3, g4]
    ssem = [s0, s1, s2, s3, s4]
    cid = lax.axis_index("c")
    sid = lax.axis_index("s")
    wid = cid * NS + sid

    z16 = jnp.zeros((16,), jnp.float32)
    o16 = jnp.ones((16,), jnp.float32)

    def fill_zrow(i, _):
        r, q = i // (D // 16), i % (D // 16)
        zrow[r, pl.ds(q * 16, 16)] = z16
        return 0

    lax.fori_loop(0, ZR * (D // 16), fill_zrow, 0)

    for o in (0, 16, CH - 16):
        onesv[pl.ds(o, 16)] = o16

    base_r = sid * RPT
    for m in range(RPT // ZR):
        pltpu.async_copy(zrow, agg_sh.at[pl.ds(base_r + m * ZR, ZR)], g0)
    for m in range(RPT // D):
        pltpu.async_copy(zrow.at[0], deg_sh.at[pl.ds(base_r + m * D, D)], g0)

    ebase = wid * EC

    def idx_group(t, p):
        for b in range(NBUF):
            pltpu.async_copy(
                src_hbm.at[pl.ds(ebase + (t * NBUF + b) * CH, CH)],
                src_idx.at[p, b], sem_i)
            pltpu.async_copy(
                dst_hbm.at[pl.ds(ebase + (t * NBUF + b) * CH, CH)],
                dst_idx.at[p, b], sem_i)

    def drain_idx():
        for _ in range(2 * NBUF):
            pltpu.make_async_copy(src_hbm.at[pl.ds(0, CH)],
                                  dst_idx.at[0, 0], sem_i).wait()

    def wait_gather(b):
        pltpu.make_async_copy(xt_hbm.at[pl.ds(0, CH)], rows.at[b],
                              gsem[b]).wait()

    def wait_scatter(b):
        pltpu.make_async_copy(xt_hbm.at[pl.ds(0, CH)], rows.at[b],
                              ssem[b]).wait()
        pltpu.make_async_copy(deg_hbm.at[pl.ds(0, CH)], onesv,
                              ssem[b]).wait()

    idx_group(0, 0)
    idx_group(1, 1)
    for m in range(RPT // ZR):
        pltpu.make_async_copy(zrow, agg_sh.at[pl.ds(base_r, ZR)], g0).wait()
    for m in range(RPT // D):
        pltpu.make_async_copy(zrow.at[0], deg_sh.at[pl.ds(base_r, D)],
                              g0).wait()
    plsc.subcore_barrier()

    def group(t, _):
        p = t % 3

        drain_idx()
        for b in range(NBUF):
            @pl.when(t > 0)
            def _(b=b):
                wait_scatter(b)

            pltpu.async_copy(xt_hbm.at[src_idx.at[p, b]], rows.at[b],
                             gsem[b])

        @pl.when(t < NGROUP - 2)
        def _():
            idx_group(t + 2, (t + 2) % 3)

        for b in range(NBUF):
            wait_gather(b)
            pltpu.async_copy(rows.at[b], agg_sh.at[dst_idx.at[p, b]],
                             ssem[b], add=True)
            pltpu.async_copy(onesv, deg_sh.at[dst_idx.at[p, b]],
                             ssem[b], add=True)
        return 0

    lax.fori_loop(0, NGROUP, group, 0)
    for b in range(NBUF):
        wait_scatter(b)
    plsc.subcore_barrier()

    pltpu.sync_copy(agg_sh.at[pl.ds(base_r, RPT)],
                    agg_hbm.at[cid, pl.ds(base_r, RPT)])
    pltpu.sync_copy(deg_sh.at[pl.ds(base_r, RPT)],
                    deg_hbm.at[pl.ds(cid * NPAD + base_r, RPT)])


@functools.cache
def _agg_call():
    return pl.kernel(
        _agg_body,
        out_type=[
            jax.ShapeDtypeStruct((NC, NPAD, D), jnp.float32),
            jax.ShapeDtypeStruct((NC * NPAD,), jnp.float32),
        ],
        mesh=plsc.VectorSubcoreMesh(core_axis_name="c", subcore_axis_name="s",
                                    num_cores=NC, num_subcores=NS),
        scratch_types=[
            pltpu.VMEM((3, NBUF, CH), jnp.int32),
            pltpu.VMEM((3, NBUF, CH), jnp.int32),
            pltpu.VMEM((NBUF, CH, D), jnp.float32),
            pltpu.VMEM((CH,), jnp.float32),
            pltpu.VMEM((ZR, D), jnp.float32),
            pltpu.SemaphoreType.DMA,
            pltpu.SemaphoreType.DMA,
            pltpu.SemaphoreType.DMA,
            pltpu.SemaphoreType.DMA,
            pltpu.SemaphoreType.DMA,
            pltpu.SemaphoreType.DMA,
            pltpu.SemaphoreType.DMA,
            pltpu.SemaphoreType.DMA,
            pltpu.SemaphoreType.DMA,
            pltpu.SemaphoreType.DMA,
            pltpu.SemaphoreType.DMA,
            pltpu.VMEM_SHARED((NPAD, D), jnp.float32),
            pltpu.VMEM_SHARED((NPAD,), jnp.float32),
        ],
    )


def _stage3_body(agg_ref, deg_ref, x_ref, k_ref, out_ref):
    agg2 = agg_ref[...]
    agg = agg2[0] + agg2[1]
    degf = deg_ref[...]
    deg8 = degf[0] + degf[1]
    r8 = lax.broadcasted_iota(jnp.int32, (BR3, 8), 0) // 128
    c8 = lax.broadcasted_iota(jnp.int32, (BR3, 8), 1)
    sel = (r8 == c8).astype(jnp.float32)
    brows = lax.dot_general(sel, deg8, (((1,), (0,)), ((), ())),
                            preferred_element_type=jnp.float32)
    l2 = lax.broadcasted_iota(jnp.int32, (BR3, D), 1)
    r2 = lax.broadcasted_iota(jnp.int32, (BR3, D), 0) % 128
    deg = jnp.sum(jnp.where(l2 == r2, brows, 0.0), axis=-1, keepdims=True)
    x = x_ref[...]
    k = k_ref[0]
    agg = agg / jnp.maximum(deg, 1.0)
    h = _proj(_expmap0(agg, 1.0), 1.0)
    xt = jax.nn.relu(_logmap0(h, 1.0))
    out1 = _proj(_expmap0(xt, 1.0), 1.0)
    out_ref[...] = _mobius_add(out1, k * x, 1.0)


def kernel(x, edge_index, dist, W, b, k):
    BR = 1000
    nblk = N // BR

    x_t, src, dst = pl.pallas_call(
        _stage1_body,
        grid=(nblk,),
        in_specs=[
            pl.BlockSpec((BR, D), lambda i: (i, 0)),
            pl.BlockSpec((D, D), lambda i: (0, 0)),
            pl.BlockSpec((1, D), lambda i: (0, 0)),
            pl.BlockSpec((2, E), lambda i: (0, 0)),
        ],
        out_specs=[
            pl.BlockSpec((BR, D), lambda i: (i, 0)),
            pl.BlockSpec((E,), lambda i: (0,)),
            pl.BlockSpec((E,), lambda i: (0,)),
        ],
        out_shape=[
            jax.ShapeDtypeStruct((N, D), jnp.float32),
            jax.ShapeDtypeStruct((E,), jnp.int32),
            jax.ShapeDtypeStruct((E,), jnp.int32),
        ],
    )(x, W, b.reshape(1, -1), edge_index)

    agg_parts, deg_flat = _agg_call()(x_t, src, dst)
    deg_parts = deg_flat.reshape(NC, NPAD // 128, 128)

    out = pl.pallas_call(
        _stage3_body,
        grid=(NPAD // BR3,),
        in_specs=[
            pl.BlockSpec((NC, BR3, D), lambda i: (0, i, 0)),
            pl.BlockSpec((NC, BR3 // 128, 128), lambda i: (0, i, 0)),
            pl.BlockSpec((BR3, D), lambda i: (i, 0)),
            pl.BlockSpec(memory_space=pltpu.SMEM),
        ],
        out_specs=pl.BlockSpec((BR3, D), lambda i: (i, 0)),
        out_shape=jax.ShapeDtypeStruct((N, D), jnp.float32),
    )(agg_parts, deg_parts, x, k)

    return (out, edge_index, dist)


# speedup vs baseline: 12.7697x; 1.0004x over previous
import functools

import jax
import jax.numpy as jnp
from jax import lax
from jax.experimental import pallas as pl
from jax.experimental.pallas import tpu as pltpu
from jax.experimental.pallas import tpu_sc as plsc

N = 10000
E = 320000
D = 128
MIN_NORM = 1e-15
EPS = 4e-3

NC = 2
NS = 16
NW = NC * NS
EC = E // NW
CH = 40
NCHUNK = EC // CH
NPAD = 10240
RPT = NPAD // NS
ZR = 32
BR3 = 1024


def _artanh(x):
    x = jnp.clip(x, -1.0 + 1e-7, 1.0 - 1e-7)
    return 0.5 * (jnp.log1p(x) - jnp.log1p(-x))


def _norm(x):
    return jnp.maximum(
        jnp.sqrt(jnp.sum(x * x, axis=-1, keepdims=True)), MIN_NORM)


def _proj(x, c):
    norm = _norm(x)
    maxnorm = (1.0 - EPS) / jnp.sqrt(c)
    return jnp.where(norm > maxnorm, x / norm * maxnorm, x)


def _expmap0(u, c):
    sc = jnp.sqrt(c)
    un = _norm(u)
    return jnp.tanh(sc * un) * u / (sc * un)


def _logmap0(p, c):
    sc = jnp.sqrt(c)
    pn = _norm(p)
    return _artanh(sc * pn) * p / (sc * pn)


def _mobius_add(x, y, c):
    x2 = jnp.sum(x * x, axis=-1, keepdims=True)
    y2 = jnp.sum(y * y, axis=-1, keepdims=True)
    xy = jnp.sum(x * y, axis=-1, keepdims=True)
    num = (1.0 + 2.0 * c * xy + c * y2) * x + (1.0 - c * x2) * y
    denom = 1.0 + 2.0 * c * xy + c * c * x2 * y2
    return num / jnp.maximum(denom, MIN_NORM)


def _stage1_body(x_ref, w_ref, b_ref, e_ref, xt_ref, src_ref, dst_ref):
    c = 1.0
    x = x_ref[...]
    w = w_ref[...]
    hb = _proj(_expmap0(b_ref[...], c), c)
    xn = _norm(x)
    mx = lax.dot_general(x, w, (((1,), (1,)), ((), ())),
                         preferred_element_type=jnp.float32)
    mxn = _norm(mx)
    res = jnp.tanh(mxn / xn * _artanh(xn)) * mx / mxn
    cond = jnp.all(mx == 0.0, axis=-1, keepdims=True)
    res = jnp.where(cond, jnp.zeros_like(res), res)
    res = _proj(res, c)
    h = _proj(_mobius_add(res, hb, c), c)
    xt_ref[...] = _logmap0(h, c)
    @pl.when(pl.program_id(0) == 0)
    def _():
        e2 = e_ref[...]
        src_ref[...] = e2[0]
        dst_ref[...] = e2[1]


NBUF = 5
NGROUP = NCHUNK // NBUF


def _agg_body(xt_hbm, src_hbm, dst_hbm, agg_hbm, deg_hbm,
              src_idx, dst_idx, rows, onesv, zrow,
              sem_i, g0, g1, g2, g3, g4, s0, s1, s2, s3, s4,
              agg_sh, deg_sh):
    gsem = [g0, g1, g2, g3, g4]
    ssem = [s0, s1, s2, s3, s4]
    cid = lax.axis_index("c")
    sid = lax.axis_index("s")
    wid = cid * NS + sid

    z16 = jnp.zeros((16,), jnp.float32)
    o16 = jnp.ones((16,), jnp.float32)

    def fill_zrow(i, _):
        r, q = i // (D // 16), i % (D // 16)
        zrow[r, pl.ds(q * 16, 16)] = z16
        return 0

    lax.fori_loop(0, ZR * (D // 16), fill_zrow, 0)

    for o in (0, 16, CH - 16):
        onesv[pl.ds(o, 16)] = o16

    base_r = sid * RPT
    for m in range(RPT // ZR):
        pltpu.async_copy(zrow, agg_sh.at[pl.ds(base_r + m * ZR, ZR)], g0)
    for m in range(RPT // D):
        pltpu.async_copy(zrow.at[0], deg_sh.at[pl.ds(base_r + m * D, D)], g0)

    ebase = wid * EC

    def idx_group(t, p):
        for b in range(NBUF):
            pltpu.async_copy(
                src_hbm.at[pl.ds(ebase + (t * NBUF + b) * CH, CH)],
                src_idx.at[p, b], sem_i)
            pltpu.async_copy(
                dst_hbm.at[pl.ds(ebase + (t * NBUF + b) * CH, CH)],
                dst_idx.at[p, b], sem_i)

    def drain_idx():
        for _ in range(2 * NBUF):
            pltpu.make_async_copy(src_hbm.at[pl.ds(0, CH)],
                                  dst_idx.at[0, 0], sem_i).wait()

    def wait_gather(b):
        pltpu.make_async_copy(xt_hbm.at[pl.ds(0, 24)],
                              rows.at[b, pl.ds(0, 24)], gsem[b]).wait()
        pltpu.make_async_copy(xt_hbm.at[pl.ds(0, 16)],
                              rows.at[b, pl.ds(24, 16)], gsem[b]).wait()

    def wait_scatter(b):
        pltpu.make_async_copy(xt_hbm.at[pl.ds(0, CH)], rows.at[b],
                              ssem[b]).wait()
        pltpu.make_async_copy(deg_hbm.at[pl.ds(0, CH)], onesv,
                              ssem[b]).wait()

    idx_group(0, 0)
    idx_group(1, 1)
    for m in range(RPT // ZR):
        pltpu.make_async_copy(zrow, agg_sh.at[pl.ds(base_r, ZR)], g0).wait()
    for m in range(RPT // D):
        pltpu.make_async_copy(zrow.at[0], deg_sh.at[pl.ds(base_r, D)],
                              g0).wait()
    plsc.subcore_barrier()

    def group(t, _):
        p = t % 3

        drain_idx()
        for b in range(NBUF):
            @pl.when(t > 0)
            def _(b=b):
                wait_scatter(b)

            pltpu.async_copy(xt_hbm.at[src_idx.at[p, b, pl.ds(0, 24)]],
                             rows.at[b, pl.ds(0, 24)], gsem[b])
            pltpu.async_copy(xt_hbm.at[src_idx.at[p, b, pl.ds(24, 16)]],
                             rows.at[b, pl.ds(24, 16)], gsem[b])

        @pl.when(t < NGROUP - 2)
        def _():
            idx_group(t + 2, (t + 2) % 3)

        for b in range(NBUF):
            wait_gather(b)
            pltpu.async_copy(rows.at[b], agg_sh.at[dst_idx.at[p, b]],
                             ssem[b], add=True)
            pltpu.async_copy(onesv, deg_sh.at[dst_idx.at[p, b]],
                             ssem[b], add=True)
        return 0

    lax.fori_loop(0, NGROUP, group, 0)
    for b in range(NBUF):
        wait_scatter(b)
    plsc.subcore_barrier()

    pltpu.sync_copy(agg_sh.at[pl.ds(base_r, RPT)],
                    agg_hbm.at[cid, pl.ds(base_r, RPT)])
    pltpu.sync_copy(deg_sh.at[pl.ds(base_r, RPT)],
                    deg_hbm.at[pl.ds(cid * NPAD + base_r, RPT)])


@functools.cache
def _agg_call():
    return pl.kernel(
        _agg_body,
        out_type=[
            jax.ShapeDtypeStruct((NC, NPAD, D), jnp.float32),
            jax.ShapeDtypeStruct((NC * NPAD,), jnp.float32),
        ],
        mesh=plsc.VectorSubcoreMesh(core_axis_name="c", subcore_axis_name="s",
                                    num_cores=NC, num_subcores=NS),
        scratch_types=[
            pltpu.VMEM((3, NBUF, CH), jnp.int32),
            pltpu.VMEM((3, NBUF, CH), jnp.int32),
            pltpu.VMEM((NBUF, CH, D), jnp.float32),
            pltpu.VMEM((CH,), jnp.float32),
            pltpu.VMEM((ZR, D), jnp.float32),
            pltpu.SemaphoreType.DMA,
            pltpu.SemaphoreType.DMA,
            pltpu.SemaphoreType.DMA,
            pltpu.SemaphoreType.DMA,
            pltpu.SemaphoreType.DMA,
            pltpu.SemaphoreType.DMA,
            pltpu.SemaphoreType.DMA,
            pltpu.SemaphoreType.DMA,
            pltpu.SemaphoreType.DMA,
            pltpu.SemaphoreType.DMA,
            pltpu.SemaphoreType.DMA,
            pltpu.VMEM_SHARED((NPAD, D), jnp.float32),
            pltpu.VMEM_SHARED((NPAD,), jnp.float32),
        ],
    )


def _stage3_body(agg_ref, deg_ref, x_ref, k_ref, out_ref):
    agg2 = agg_ref[...]
    agg = agg2[0] + agg2[1]
    degf = deg_ref[...]
    deg8 = degf[0] + degf[1]
    r8 = lax.broadcasted_iota(jnp.int32, (BR3, 8), 0) // 128
    c8 = lax.broadcasted_iota(jnp.int32, (BR3, 8), 1)
    sel = (r8 == c8).astype(jnp.float32)
    brows = lax.dot_general(sel, deg8, (((1,), (0,)), ((), ())),
                            preferred_element_type=jnp.float32)
    l2 = lax.broadcasted_iota(jnp.int32, (BR3, D), 1)
    r2 = lax.broadcasted_iota(jnp.int32, (BR3, D), 0) % 128
    deg = jnp.sum(jnp.where(l2 == r2, brows, 0.0), axis=-1, keepdims=True)
    x = x_ref[...]
    k = k_ref[0]
    agg = agg / jnp.maximum(deg, 1.0)
    h = _proj(_expmap0(agg, 1.0), 1.0)
    xt = jax.nn.relu(_logmap0(h, 1.0))
    out1 = _proj(_expmap0(xt, 1.0), 1.0)
    out_ref[...] = _mobius_add(out1, k * x, 1.0)


def kernel(x, edge_index, dist, W, b, k):
    BR = 1000
    nblk = N // BR

    x_t, src, dst = pl.pallas_call(
        _stage1_body,
        grid=(nblk,),
        in_specs=[
            pl.BlockSpec((BR, D), lambda i: (i, 0)),
            pl.BlockSpec((D, D), lambda i: (0, 0)),
            pl.BlockSpec((1, D), lambda i: (0, 0)),
            pl.BlockSpec((2, E), lambda i: (0, 0)),
        ],
        out_specs=[
            pl.BlockSpec((BR, D), lambda i: (i, 0)),
            pl.BlockSpec((E,), lambda i: (0,)),
            pl.BlockSpec((E,), lambda i: (0,)),
        ],
        out_shape=[
            jax.ShapeDtypeStruct((N, D), jnp.float32),
            jax.ShapeDtypeStruct((E,), jnp.int32),
            jax.ShapeDtypeStruct((E,), jnp.int32),
        ],
    )(x, W, b.reshape(1, -1), edge_index)

    agg_parts, deg_flat = _agg_call()(x_t, src, dst)
    deg_parts = deg_flat.reshape(NC, NPAD // 128, 128)

    out = pl.pallas_call(
        _stage3_body,
        grid=(NPAD // BR3,),
        in_specs=[
            pl.BlockSpec((NC, BR3, D), lambda i: (0, i, 0)),
            pl.BlockSpec((NC, BR3 // 128, 128), lambda i: (0, i, 0)),
            pl.BlockSpec((BR3, D), lambda i: (i, 0)),
            pl.BlockSpec(memory_space=pltpu.SMEM),
        ],
        out_specs=pl.BlockSpec((BR3, D), lambda i: (i, 0)),
        out_shape=jax.ShapeDtypeStruct((N, D), jnp.float32),
    )(agg_parts, deg_parts, x, k)

    return (out, edge_index, dist)
